# Initial kernel scaffold; baseline (speedup 1.0000x reference)
import jax
import jax.numpy as jnp
from jax.experimental import pallas as pl


def kernel(x, pe):
    raise NotImplementedError("write your pallas kernel here")


# speedup vs baseline: 1.4165x; 1.4165x over previous
import jax
import jax.numpy as jnp
from jax import lax
from jax.experimental import pallas as pl
from jax.experimental.pallas import tpu as pltpu, tpu_sc as plsc

DIM = 128
MAX_LEN = 10000
BATCH = 16384

_INFO = plsc.get_sparse_core_info()
_NC, _NS, _L = _INFO.num_cores, _INFO.num_subcores, _INFO.num_lanes
_NW = _NC * _NS
_B_PER_W = BATCH // _NW
_CHUNK = 128
_NCHUNK = _B_PER_W // _CHUNK


def _body(x_hbm, pe_hbm, out_hbm, x_v, idx_v, rows_v, sem):
    wid = lax.axis_index("s") * _NC + lax.axis_index("c")
    base = wid * _B_PER_W

    pltpu.sync_copy(x_hbm.at[pl.ds(base, _B_PER_W)], x_v)

    for j in range(_NCHUNK):
        for k in range(_CHUNK // _L):
            xv = x_v[pl.ds(j * _CHUNK + k * _L, _L)]
            iv = (xv * 1000.0).astype(jnp.int32)
            iv = jnp.minimum(jnp.maximum(iv, 0), MAX_LEN - 1)
            idx_v[j, pl.ds(k * _L, _L)] = iv

    copies = [
        pltpu.make_async_copy(
            pe_hbm.at[idx_v.at[j]],
            rows_v.at[pl.ds(j * _CHUNK, _CHUNK)],
            sem,
        )
        for j in range(_NCHUNK)
    ]
    for c in copies:
        c.start()
    for c in copies:
        c.wait()

    pltpu.sync_copy(rows_v, out_hbm.at[pl.ds(base, _B_PER_W)])


def kernel(x, pe):
    mesh = plsc.VectorSubcoreMesh(core_axis_name="c", subcore_axis_name="s")
    f = pl.kernel(
        _body,
        mesh=mesh,
        out_type=jax.ShapeDtypeStruct((BATCH, DIM), jnp.float32),
        scratch_types=[
            pltpu.VMEM((_B_PER_W,), jnp.float32),
            pltpu.VMEM((_NCHUNK, _CHUNK), jnp.int32),
            pltpu.VMEM((_B_PER_W, DIM), jnp.float32),
            pltpu.SemaphoreType.DMA,
        ],
    )
    return f(x, pe)


# speedup vs baseline: 1.4571x; 1.0287x over previous
import jax
import jax.numpy as jnp
from jax import lax
from jax.experimental import pallas as pl
from jax.experimental.pallas import tpu as pltpu, tpu_sc as plsc

DIM = 128
MAX_LEN = 10000
BATCH = 16384

_INFO = plsc.get_sparse_core_info()
_NC, _NS, _L = _INFO.num_cores, _INFO.num_subcores, _INFO.num_lanes
_NW = _NC * _NS
_B_PER_W = BATCH // _NW
_CHUNK = 128
_NCHUNK = _B_PER_W // _CHUNK


def _body(x_hbm, pe_hbm, out_hbm, x_v, idx_v, rows_v, gsem, wsem):
    wid = lax.axis_index("s") * _NC + lax.axis_index("c")
    base = wid * _B_PER_W

    pltpu.sync_copy(x_hbm.at[pl.ds(base, _B_PER_W)], x_v)

    for j in range(_NCHUNK):
        for k in range(_CHUNK // _L):
            xv = x_v[pl.ds(j * _CHUNK + k * _L, _L)]
            iv = (xv * 1000.0).astype(jnp.int32)
            iv = jnp.minimum(jnp.maximum(iv, 0), MAX_LEN - 1)
            idx_v[j, pl.ds(k * _L, _L)] = iv

    gathers = [
        pltpu.make_async_copy(
            pe_hbm.at[idx_v.at[j]],
            rows_v.at[pl.ds(j * _CHUNK, _CHUNK)],
            gsem,
        )
        for j in range(_NCHUNK)
    ]
    writes = [
        pltpu.make_async_copy(
            rows_v.at[pl.ds(j * _CHUNK, _CHUNK)],
            out_hbm.at[pl.ds(base + j * _CHUNK, _CHUNK)],
            wsem,
        )
        for j in range(_NCHUNK)
    ]
    for g in gathers:
        g.start()
    for j in range(_NCHUNK):
        gathers[j].wait()
        writes[j].start()
    for w in writes:
        w.wait()


def kernel(x, pe):
    mesh = plsc.VectorSubcoreMesh(core_axis_name="c", subcore_axis_name="s")
    f = pl.kernel(
        _body,
        mesh=mesh,
        out_type=jax.ShapeDtypeStruct((BATCH, DIM), jnp.float32),
        scratch_types=[
            pltpu.VMEM((_B_PER_W,), jnp.float32),
            pltpu.VMEM((_NCHUNK, _CHUNK), jnp.int32),
            pltpu.VMEM((_B_PER_W, DIM), jnp.float32),
            pltpu.SemaphoreType.DMA,
            pltpu.SemaphoreType.DMA,
        ],
    )
    return f(x, pe)
